# baseline (device time: 155174 ns/iter reference)
import jax
import jax.numpy as jnp
from jax import lax
from jax.experimental import pallas as pl
from jax.experimental.pallas import tpu as pltpu

N_Z = 4
MESH = pl.DeviceIdType.MESH


def kernel(x):
    x16 = x.astype(jnp.bfloat16)
    m_per, n = x16.shape
    Q = m_per // 4
    H = Q // 2

    def body(x_hbm, out_hbm, qmine, qx, qy, qdx, qdy,
             z_ssem, z_rsem, x_ssem, x_rsem, y_ssem, y_rsem,
             rx_ssem, rx_rsem, ry_ssem, ry_rsem, asm_sems, in_sems):
        my_x = lax.axis_index("x")
        my_y = lax.axis_index("y")
        my_z = lax.axis_index("z")
        qi = 2 * my_x + my_y
        qx_idx = 2 * (1 - my_x) + my_y
        qy_idx = 2 * my_x + (1 - my_y)
        qd_idx = 2 * (1 - my_x) + (1 - my_y)
        z_left = (my_z - 1) % N_Z
        z_right = (my_z + 1) % N_Z
        xp = (1 - my_x, my_y, my_z)
        yp = (my_x, 1 - my_y, my_z)

        in_q = [
            pltpu.make_async_copy(
                x_hbm.at[pl.ds(qi * Q + j * H, H), :], qmine.at[0, j],
                in_sems.at[j])
            for j in range(2)
        ]
        for c in in_q:
            c.start()
        own = pltpu.make_async_copy(
            x_hbm, out_hbm.at[pl.ds(my_z * m_per, m_per), :], in_sems.at[2])
        own.start()

        barrier_sem = pltpu.get_barrier_semaphore()
        for dev in [(my_x, my_y, z_left), (my_x, my_y, z_right), xp, yp]:
            pl.semaphore_signal(barrier_sem, inc=1, device_id=dev,
                                device_id_type=MESH)
        pl.semaphore_wait(barrier_sem, 4)

        for c in in_q:
            c.wait()

        def z_fwd(r, j):
            return pltpu.make_async_remote_copy(
                src_ref=qmine.at[r, j], dst_ref=qmine.at[r + 1, j],
                send_sem=z_ssem.at[2 * r + j], recv_sem=z_rsem.at[2 * r + j],
                device_id=(my_x, my_y, z_right), device_id_type=MESH)

        def plane(r, j):
            s = r - 1
            px = pltpu.make_async_remote_copy(
                src_ref=qmine.at[r, j], dst_ref=qx.at[s, j],
                send_sem=x_ssem.at[2 * s + j], recv_sem=x_rsem.at[2 * s + j],
                device_id=xp, device_id_type=MESH)
            py = pltpu.make_async_remote_copy(
                src_ref=qmine.at[r, j], dst_ref=qy.at[s, j],
                send_sem=y_ssem.at[2 * s + j], recv_sem=y_rsem.at[2 * s + j],
                device_id=yp, device_id_type=MESH)
            px.start()
            py.start()
            return px, py

        pending = []
        zf = {}
        for j in range(2):
            zf[(0, j)] = z_fwd(0, j)
            zf[(0, j)].start()
            pending.append(zf[(0, j)])
        prev_asm = [own]

        def finish_round(s, px0, px1, py0, py1):
            px1.wait_recv()
            ry = pltpu.make_async_remote_copy(
                src_ref=qx.at[s, 1], dst_ref=qdy.at[s],
                send_sem=ry_ssem.at[s], recv_sem=ry_rsem.at[s],
                device_id=yp, device_id_type=MESH)
            ry.start()
            pending.append(ry)
            py0.wait_recv()
            rx = pltpu.make_async_remote_copy(
                src_ref=qy.at[s, 0], dst_ref=qdx.at[s],
                send_sem=rx_ssem.at[s], recv_sem=rx_rsem.at[s],
                device_id=xp, device_id_type=MESH)
            rx.start()
            pending.append(rx)
            px0.wait_recv()
            py1.wait_recv()
            rx.wait_recv()
            ry.wait_recv()

            for c in prev_asm:
                c.wait()
            prev_asm.clear()
            base = ((my_z - (s + 1)) % N_Z) * m_per
            for src, row0, sem in [
                (qmine.at[s + 1, 0], base + qi * Q, 0),
                (qmine.at[s + 1, 1], base + qi * Q + H, 1),
                (qx.at[s, 0], base + qx_idx * Q, 2),
                (qx.at[s, 1], base + qx_idx * Q + H, 3),
                (qy.at[s, 0], base + qy_idx * Q, 4),
                (qy.at[s, 1], base + qy_idx * Q + H, 5),
                (qdx.at[s], base + qd_idx * Q, 6),
                (qdy.at[s], base + qd_idx * Q + H, 7),
            ]:
                cp = pltpu.make_async_copy(
                    src, out_hbm.at[pl.ds(row0, H), :], asm_sems.at[sem])
                cp.start()
                prev_asm.append(cp)

        plane_prev = None
        for r in range(1, N_Z):
            pp = []
            for j in range(2):
                zf[(r - 1, j)].wait_recv()
                if r < N_Z - 1:
                    zf[(r, j)] = z_fwd(r, j)
                    zf[(r, j)].start()
                    pending.append(zf[(r, j)])
                px, py = plane(r, j)
                pending += [px, py]
                pp += [px, py]
            if plane_prev is not None:
                finish_round(*plane_prev)
            plane_prev = (r - 1, pp[0], pp[2], pp[1], pp[3])

        finish_round(*plane_prev)
        for c in prev_asm:
            c.wait()
        for d in pending:
            d.wait_send()

    return pl.pallas_call(
        body,
        out_shape=jax.ShapeDtypeStruct((N_Z * m_per, n), jnp.bfloat16),
        in_specs=[pl.BlockSpec(memory_space=pltpu.MemorySpace.HBM)],
        out_specs=pl.BlockSpec(memory_space=pltpu.MemorySpace.HBM),
        scratch_shapes=[
            pltpu.VMEM((N_Z, 2, H, n), jnp.bfloat16),
            pltpu.VMEM((N_Z - 1, 2, H, n), jnp.bfloat16),
            pltpu.VMEM((N_Z - 1, 2, H, n), jnp.bfloat16),
            pltpu.VMEM((N_Z - 1, H, n), jnp.bfloat16),
            pltpu.VMEM((N_Z - 1, H, n), jnp.bfloat16),
            pltpu.SemaphoreType.DMA((2 * (N_Z - 1),)),
            pltpu.SemaphoreType.DMA((2 * (N_Z - 1),)),
            pltpu.SemaphoreType.DMA((2 * (N_Z - 1),)),
            pltpu.SemaphoreType.DMA((2 * (N_Z - 1),)),
            pltpu.SemaphoreType.DMA((2 * (N_Z - 1),)),
            pltpu.SemaphoreType.DMA((2 * (N_Z - 1),)),
            pltpu.SemaphoreType.DMA((N_Z - 1,)),
            pltpu.SemaphoreType.DMA((N_Z - 1,)),
            pltpu.SemaphoreType.DMA((N_Z - 1,)),
            pltpu.SemaphoreType.DMA((N_Z - 1,)),
            pltpu.SemaphoreType.DMA((8,)),
            pltpu.SemaphoreType.DMA((3,)),
        ],
        compiler_params=pltpu.CompilerParams(collective_id=0),
    )(x16)


# device time: 151763 ns/iter; 1.0225x vs baseline; 1.0225x over previous
import jax
import jax.numpy as jnp
from jax import lax
from jax.experimental import pallas as pl
from jax.experimental.pallas import tpu as pltpu

N_Z = 4
S = 4
MESH = pl.DeviceIdType.MESH


def kernel(x):
    x16 = x.astype(jnp.bfloat16)
    m_per, n = x16.shape
    Q = m_per // 4
    H = Q // 2
    Hs = Q // S

    def body(x_hbm, out_hbm, qmine, qx, qy, qdx, qdy,
             z_ssem, z_rsem, x_ssem, x_rsem, y_ssem, y_rsem,
             rx_ssem, rx_rsem, ry_ssem, ry_rsem, asm_sems, in_sems):
        my_x = lax.axis_index("x")
        my_y = lax.axis_index("y")
        my_z = lax.axis_index("z")
        qi = 2 * my_x + my_y
        qx_idx = 2 * (1 - my_x) + my_y
        qy_idx = 2 * my_x + (1 - my_y)
        qd_idx = 2 * (1 - my_x) + (1 - my_y)
        z_left = (my_z - 1) % N_Z
        z_right = (my_z + 1) % N_Z
        xp = (1 - my_x, my_y, my_z)
        yp = (my_x, 1 - my_y, my_z)

        in_q = [
            pltpu.make_async_copy(
                x_hbm.at[pl.ds(qi * Q + j * Hs, Hs), :], qmine.at[0, j],
                in_sems.at[j])
            for j in range(S)
        ]
        for c in in_q:
            c.start()
        own = pltpu.make_async_copy(
            x_hbm, out_hbm.at[pl.ds(my_z * m_per, m_per), :], in_sems.at[S])
        own.start()

        barrier_sem = pltpu.get_barrier_semaphore()
        for dev in [(my_x, my_y, z_left), (my_x, my_y, z_right), xp, yp]:
            pl.semaphore_signal(barrier_sem, inc=1, device_id=dev,
                                device_id_type=MESH)
        pl.semaphore_wait(barrier_sem, 4)

        for c in in_q:
            c.wait()

        def z_fwd(r, j):
            return pltpu.make_async_remote_copy(
                src_ref=qmine.at[r, j], dst_ref=qmine.at[r + 1, j],
                send_sem=z_ssem.at[S * r + j], recv_sem=z_rsem.at[S * r + j],
                device_id=(my_x, my_y, z_right), device_id_type=MESH)

        pending = []
        zf = {}
        for j in range(S):
            zf[(0, j)] = z_fwd(0, j)
            zf[(0, j)].start()
            pending.append(zf[(0, j)])
        prev_asm = [own]

        def finish_round(s, pxs, pys):
            relays = []
            for k in range(S // 2, S):
                pxs[k].wait_recv()
                ry = pltpu.make_async_remote_copy(
                    src_ref=qx.at[s, k], dst_ref=qdy.at[s, k - S // 2],
                    send_sem=ry_ssem.at[(S // 2) * s + k - S // 2],
                    recv_sem=ry_rsem.at[(S // 2) * s + k - S // 2],
                    device_id=yp, device_id_type=MESH)
                ry.start()
                relays.append(ry)
            for k in range(S // 2):
                pys[k].wait_recv()
                rx = pltpu.make_async_remote_copy(
                    src_ref=qy.at[s, k], dst_ref=qdx.at[s, k],
                    send_sem=rx_ssem.at[(S // 2) * s + k],
                    recv_sem=rx_rsem.at[(S // 2) * s + k],
                    device_id=xp, device_id_type=MESH)
                rx.start()
                relays.append(rx)
            pending.extend(relays)
            for k in range(S // 2):
                pxs[k].wait_recv()
            for k in range(S // 2, S):
                pys[k].wait_recv()
            for d in relays:
                d.wait_recv()

            for c in prev_asm:
                c.wait()
            prev_asm.clear()
            base = ((my_z - (s + 1)) % N_Z) * m_per
            copies = []
            for j in range(S):
                copies.append((qmine.at[s + 1, j], base + qi * Q + j * Hs))
                copies.append((qx.at[s, j], base + qx_idx * Q + j * Hs))
                copies.append((qy.at[s, j], base + qy_idx * Q + j * Hs))
            for k in range(S // 2):
                copies.append((qdx.at[s, k], base + qd_idx * Q + k * Hs))
                copies.append(
                    (qdy.at[s, k], base + qd_idx * Q + H + k * Hs))
            for sem, (src, row0) in enumerate(copies):
                cp = pltpu.make_async_copy(
                    src, out_hbm.at[pl.ds(row0, Hs), :], asm_sems.at[sem])
                cp.start()
                prev_asm.append(cp)

        plane_prev = None
        for r in range(1, N_Z):
            s = r - 1
            pxs, pys = [], []
            for j in range(S):
                zf[(r - 1, j)].wait_recv()
                if r < N_Z - 1:
                    zf[(r, j)] = z_fwd(r, j)
                    zf[(r, j)].start()
                    pending.append(zf[(r, j)])
                px = pltpu.make_async_remote_copy(
                    src_ref=qmine.at[r, j], dst_ref=qx.at[s, j],
                    send_sem=x_ssem.at[S * s + j],
                    recv_sem=x_rsem.at[S * s + j],
                    device_id=xp, device_id_type=MESH)
                py = pltpu.make_async_remote_copy(
                    src_ref=qmine.at[r, j], dst_ref=qy.at[s, j],
                    send_sem=y_ssem.at[S * s + j],
                    recv_sem=y_rsem.at[S * s + j],
                    device_id=yp, device_id_type=MESH)
                px.start()
                py.start()
                pending += [px, py]
                pxs.append(px)
                pys.append(py)
            if plane_prev is not None:
                finish_round(*plane_prev)
            plane_prev = (s, pxs, pys)

        finish_round(*plane_prev)
        for c in prev_asm:
            c.wait()
        for d in pending:
            d.wait_send()

    return pl.pallas_call(
        body,
        out_shape=jax.ShapeDtypeStruct((N_Z * m_per, n), jnp.bfloat16),
        in_specs=[pl.BlockSpec(memory_space=pltpu.MemorySpace.HBM)],
        out_specs=pl.BlockSpec(memory_space=pltpu.MemorySpace.HBM),
        scratch_shapes=[
            pltpu.VMEM((N_Z, S, Hs, n), jnp.bfloat16),
            pltpu.VMEM((N_Z - 1, S, Hs, n), jnp.bfloat16),
            pltpu.VMEM((N_Z - 1, S, Hs, n), jnp.bfloat16),
            pltpu.VMEM((N_Z - 1, S // 2, Hs, n), jnp.bfloat16),
            pltpu.VMEM((N_Z - 1, S // 2, Hs, n), jnp.bfloat16),
            pltpu.SemaphoreType.DMA((S * (N_Z - 1),)),
            pltpu.SemaphoreType.DMA((S * (N_Z - 1),)),
            pltpu.SemaphoreType.DMA((S * (N_Z - 1),)),
            pltpu.SemaphoreType.DMA((S * (N_Z - 1),)),
            pltpu.SemaphoreType.DMA((S * (N_Z - 1),)),
            pltpu.SemaphoreType.DMA((S * (N_Z - 1),)),
            pltpu.SemaphoreType.DMA(((S // 2) * (N_Z - 1),)),
            pltpu.SemaphoreType.DMA(((S // 2) * (N_Z - 1),)),
            pltpu.SemaphoreType.DMA(((S // 2) * (N_Z - 1),)),
            pltpu.SemaphoreType.DMA(((S // 2) * (N_Z - 1),)),
            pltpu.SemaphoreType.DMA((4 * S,)),
            pltpu.SemaphoreType.DMA((S + 1,)),
        ],
        compiler_params=pltpu.CompilerParams(collective_id=0),
    )(x16)
